# Initial kernel scaffold; baseline (speedup 1.0000x reference)
import jax
import jax.numpy as jnp
from jax.experimental import pallas as pl


def kernel(x, edge_index, batch, W1_l, b1, W1_r, W2_l, b2, W2_r):
    raise NotImplementedError("write your pallas kernel here")


# speedup vs baseline: 4.3377x; 4.3377x over previous
import functools

import jax
import jax.numpy as jnp
from jax import lax
from jax.experimental import pallas as pl
from jax.experimental.pallas import tpu as pltpu
from jax.experimental.pallas import tpu_sc as plsc

N = 10000
NP = 10240
D = 128
G = 64
NC, NS = 2, 16
NW = NC * NS
B = 128
ROWS_PER_W = NP // NS
R = 1024
NBLK = NP // R

@functools.cache
def _mesh():
    return plsc.VectorSubcoreMesh(
        core_axis_name="c", subcore_axis_name="s",
        num_cores=NC, num_subcores=NS)


def _sc_spmm_body(table, src_i, dst_i, out_acc,
                  srcv, dstv, rows, zbuf, sem, acc_sh):
    cid = lax.axis_index("c")
    sid = lax.axis_index("s")
    wid = cid * NS + sid
    base = sid * ROWS_PER_W
    zero16 = jnp.zeros((16,), jnp.float32)

    def zrow(i, c):
        for k in range(8):
            zbuf[i, pl.ds(k * 16, 16)] = zero16
        return c
    lax.fori_loop(0, 64, zrow, 0)

    def zacc(k, c):
        pltpu.sync_copy(zbuf, acc_sh.at[pl.ds(base + k * 64, 64)])
        return c
    lax.fori_loop(0, ROWS_PER_W // 64, zacc, 0)

    pltpu.sync_copy(src_i.at[wid], srcv)
    pltpu.sync_copy(dst_i.at[wid], dstv)
    plsc.subcore_barrier()

    nb = srcv.shape[0]

    def step(j, c):
        pltpu.async_copy(table.at[srcv.at[j]], rows, sem).wait()
        pltpu.sync_copy(rows, acc_sh.at[dstv.at[j]], add=True)
        return c
    lax.fori_loop(0, nb, step, 0)

    plsc.subcore_barrier()

    def cout(k, c):
        pltpu.sync_copy(acc_sh.at[pl.ds(base + k * 64, 64)], zbuf)
        pltpu.sync_copy(zbuf, out_acc.at[cid, pl.ds(base + k * 64, 64)])
        return c
    lax.fori_loop(0, ROWS_PER_W // 64, cout, 0)


def _make_sc_spmm(nb):
    return pl.kernel(
        _sc_spmm_body,
        out_type=[jax.ShapeDtypeStruct((NC, NP, D), jnp.float32)],
        mesh=_mesh(),
        scratch_types=[
            pltpu.VMEM((nb, B), jnp.int32),
            pltpu.VMEM((nb, B), jnp.int32),
            pltpu.VMEM((B, D), jnp.float32),
            pltpu.VMEM((64, D), jnp.float32),
            pltpu.SemaphoreType.DMA,
            pltpu.VMEM_SHARED((NP, D), jnp.float32),
        ],
    )


def _sc_deg_body(dst_i, out_deg, dstv, onesv, zbuf, deg_sh):
    cid = lax.axis_index("c")
    sid = lax.axis_index("s")
    wid = cid * NS + sid
    base = sid * ROWS_PER_W
    zero16 = jnp.zeros((16,), jnp.float32)
    ones16 = jnp.ones((16,), jnp.float32)

    def orow(i, c):
        for k in range(8):
            onesv[i, pl.ds(k * 16, 16)] = ones16
        return c
    lax.fori_loop(0, B, orow, 0)

    def zrow(i, c):
        for k in range(8):
            zbuf[i, pl.ds(k * 16, 16)] = zero16
        return c
    lax.fori_loop(0, 64, zrow, 0)

    def zdeg(k, c):
        pltpu.sync_copy(zbuf, deg_sh.at[pl.ds(base + k * 64, 64)])
        return c
    lax.fori_loop(0, ROWS_PER_W // 64, zdeg, 0)

    pltpu.sync_copy(dst_i.at[wid], dstv)
    plsc.subcore_barrier()

    nb = dstv.shape[0]

    def step(j, c):
        pltpu.sync_copy(onesv, deg_sh.at[dstv.at[j]], add=True)
        return c
    lax.fori_loop(0, nb, step, 0)

    plsc.subcore_barrier()

    def cout(k, c):
        pltpu.sync_copy(deg_sh.at[pl.ds(base + k * 64, 64)], zbuf)
        pltpu.sync_copy(zbuf, out_deg.at[cid, pl.ds(base + k * 64, 64)])
        return c
    lax.fori_loop(0, ROWS_PER_W // 64, cout, 0)


def _make_sc_deg(nb):
    return pl.kernel(
        _sc_deg_body,
        out_type=[jax.ShapeDtypeStruct((NC, NP, D), jnp.float32)],
        mesh=_mesh(),
        scratch_types=[
            pltpu.VMEM((nb, B), jnp.int32),
            pltpu.VMEM((B, D), jnp.float32),
            pltpu.VMEM((64, D), jnp.float32),
            pltpu.VMEM_SHARED((NP, D), jnp.float32),
        ],
    )


def _tc1_body(x_ref, wl_ref, wr_ref, ol_ref, or_ref):
    xv = x_ref[...]
    ol_ref[...] = jnp.dot(xv, wl_ref[...], preferred_element_type=jnp.float32)
    or_ref[...] = jnp.dot(xv, wr_ref[...], preferred_element_type=jnp.float32)


def _tc2_body(acc_ref, deg_ref, xwr_ref, b1_ref, w2l_ref, w2r_ref,
              ol_ref, or_ref):
    deg = jnp.maximum(deg_ref[0, :, 0:1] + deg_ref[1, :, 0:1], 1.0)
    aggr = (acc_ref[0, :, :] + acc_ref[1, :, :]) / deg
    h = jnp.maximum(aggr + b1_ref[...] + xwr_ref[...], 0.0)
    ol_ref[...] = jnp.dot(h, w2l_ref[...], preferred_element_type=jnp.float32)
    or_ref[...] = jnp.dot(h, w2r_ref[...], preferred_element_type=jnp.float32)


def _tc3_body(acc_ref, deg_ref, hwr_ref, b2_ref, batch_ref,
              hmean_ref, logp_ref, pooled_ref):
    i = pl.program_id(0)
    deg = jnp.maximum(deg_ref[0, :, 0:1] + deg_ref[1, :, 0:1], 1.0)
    h2 = (acc_ref[0, :, :] + acc_ref[1, :, :]) / deg + b2_ref[...] + hwr_ref[...]
    m = jnp.max(h2, axis=1, keepdims=True)
    lse = jnp.log(jnp.sum(jnp.exp(h2 - m), axis=1, keepdims=True)) + m
    logp_ref[...] = h2 - lse

    @pl.when(i == 0)
    def _init():
        pooled_ref[...] = jnp.full((G, D), -jnp.inf, jnp.float32)

    b = batch_ref[...]
    rows = [jnp.max(jnp.where(b == g, h2, -jnp.inf), axis=0) for g in range(G)]
    pooled_ref[...] = jnp.maximum(pooled_ref[...], jnp.stack(rows))

    @pl.when(i == NBLK - 1)
    def _fin():
        hmean_ref[...] = jnp.mean(pooled_ref[...], axis=0, keepdims=True)


def kernel(x, edge_index, batch, W1_l, b1, W1_r, W2_l, b2, W2_r):
    E = edge_index.shape[1]
    nb = -(-E // (NW * B))
    ep = NW * nb * B
    src = jnp.pad(edge_index[0], (0, ep - E), constant_values=NP - 1)
    dst = jnp.pad(edge_index[1], (0, ep - E), constant_values=NP - 1)
    src = src.reshape(NW, nb, B)
    dst = dst.reshape(NW, nb, B)
    x_pad = jnp.pad(x, ((0, NP - N), (0, 0)))
    batch2d = jnp.pad(batch, (0, NP - N), constant_values=G).reshape(NP, 1)
    b1r = b1.reshape(1, D)
    b2r = b2.reshape(1, D)

    tc1 = pl.pallas_call(
        _tc1_body,
        grid=(NBLK,),
        in_specs=[
            pl.BlockSpec((R, D), lambda i: (i, 0)),
            pl.BlockSpec((D, D), lambda i: (0, 0)),
            pl.BlockSpec((D, D), lambda i: (0, 0)),
        ],
        out_specs=[
            pl.BlockSpec((R, D), lambda i: (i, 0)),
            pl.BlockSpec((R, D), lambda i: (i, 0)),
        ],
        out_shape=[
            jax.ShapeDtypeStruct((NP, D), jnp.float32),
            jax.ShapeDtypeStruct((NP, D), jnp.float32),
        ],
    )
    xW1l, xW1r = tc1(x_pad, W1_l, W1_r)

    sc1 = _make_sc_spmm(nb)
    (acc1,) = sc1(xW1l, src, dst)
    (degS,) = _make_sc_deg(nb)(dst)

    tc2 = pl.pallas_call(
        _tc2_body,
        grid=(NBLK,),
        in_specs=[
            pl.BlockSpec((NC, R, D), lambda i: (0, i, 0)),
            pl.BlockSpec((NC, R, D), lambda i: (0, i, 0)),
            pl.BlockSpec((R, D), lambda i: (i, 0)),
            pl.BlockSpec((1, D), lambda i: (0, 0)),
            pl.BlockSpec((D, D), lambda i: (0, 0)),
            pl.BlockSpec((D, D), lambda i: (0, 0)),
        ],
        out_specs=[
            pl.BlockSpec((R, D), lambda i: (i, 0)),
            pl.BlockSpec((R, D), lambda i: (i, 0)),
        ],
        out_shape=[
            jax.ShapeDtypeStruct((NP, D), jnp.float32),
            jax.ShapeDtypeStruct((NP, D), jnp.float32),
        ],
    )
    hW2l, hW2r = tc2(acc1, degS, xW1r, b1r, W2_l, W2_r)

    sc2 = _make_sc_spmm(nb)
    (acc2,) = sc2(hW2l, src, dst)

    tc3 = pl.pallas_call(
        _tc3_body,
        grid=(NBLK,),
        in_specs=[
            pl.BlockSpec((NC, R, D), lambda i: (0, i, 0)),
            pl.BlockSpec((NC, R, D), lambda i: (0, i, 0)),
            pl.BlockSpec((R, D), lambda i: (i, 0)),
            pl.BlockSpec((1, D), lambda i: (0, 0)),
            pl.BlockSpec((R, 1), lambda i: (i, 0)),
        ],
        out_specs=[
            pl.BlockSpec((1, D), lambda i: (0, 0)),
            pl.BlockSpec((R, D), lambda i: (i, 0)),
        ],
        out_shape=[
            jax.ShapeDtypeStruct((1, D), jnp.float32),
            jax.ShapeDtypeStruct((NP, D), jnp.float32),
        ],
        scratch_shapes=[pltpu.VMEM((G, D), jnp.float32)],
    )
    hmean, logp = tc3(acc2, degS, hW2r, b2r, batch2d)
    return (hmean, logp[:N])
